# Initial kernel scaffold; baseline (speedup 1.0000x reference)
import jax
import jax.numpy as jnp
from jax.experimental import pallas as pl


def kernel(info_bits, noise):
    raise NotImplementedError("write your pallas kernel here")


# speedup vs baseline: 12.5806x; 12.5806x over previous
import numpy as np
import jax
import jax.numpy as jnp
from jax.experimental import pallas as pl

_N = 4096
_K = 2048
_EBNO_DB = 2.0
_THRESH = 0.9
_BATCH = 1024
_ROWS = 128
_QG = _N // 128


def _code_construction():
    z = np.array([0.5], dtype=np.float64)
    while z.size < _N:
        z = np.concatenate([2.0 * z - z * z, z * z])
    info_set = np.sort(np.argsort(z)[:_K])
    return info_set, z


def _build_constants():
    info_set, z = _code_construction()
    j = np.arange(_N)
    gk = ((j[None, :] & ~info_set[:, None]) == 0).astype(np.float32)
    l = np.arange(128)
    m1 = ((l[None, :] & ~l[:, None]) == 0).astype(np.float32)
    r = (1.0 - z).astype(np.float32)
    frozen = np.ones((_N,), np.float32)
    frozen[info_set] = 0.0
    is_info = 1.0 - frozen
    consts = np.zeros((8, _N), np.float32)
    consts[0, :] = r
    consts[1, :] = is_info
    return gk, m1, consts


_GK_NP, _M1_NP, _CONSTS_NP = _build_constants()
_SIGMA = float(np.sqrt(1.0 / (2.0 * (_K / _N) * 10.0 ** (_EBNO_DB / 10.0))))


def _polar_metrics_kernel(bits_ref, noise_ref, gk_ref, m1_ref, c_ref, out_ref):
    i = pl.program_id(0)

    @pl.when(i == 0)
    def _init():
        out_ref[...] = jnp.zeros_like(out_ref)

    rvec = c_ref[0:1, :]
    ivec = c_ref[1:2, :]

    bits_bf = bits_ref[...].astype(jnp.bfloat16)
    s_full = jnp.dot(bits_bf, gk_ref[...], preferred_element_type=jnp.float32)
    x_int = jnp.bitwise_and(s_full.astype(jnp.int32), 1)

    inv2s2 = 2.0 / (_SIGMA * _SIGMA)
    c_mod = (1 - 2 * x_int).astype(jnp.float32)
    y = c_mod + _SIGMA * noise_ref[...]
    abs_llr = inv2s2 * jnp.abs(y)
    sig = 1.0 / (1.0 + jnp.exp(-abs_llr))
    p_u = 0.5 * (rvec + sig)
    maskf = jnp.where(p_u < _THRESH, 0.0, 1.0) * ivec

    xhat = (y < 0).astype(jnp.int32)
    e_bf = jnp.bitwise_xor(xhat, x_int).astype(jnp.bfloat16)

    m1 = m1_ref[...].astype(jnp.bfloat16)
    t = [
        jnp.dot(e_bf[:, 128 * q:128 * (q + 1)], m1,
                preferred_element_type=jnp.float32)
        for q in range(_QG)
    ]
    for s in (1, 2, 4, 8, 16):
        for q in range(_QG):
            if q & s == 0:
                t[q] = t[q] + t[q + s]

    s_mask = jnp.sum(maskf)
    s_em = jnp.float32(0.0)
    row_em = jnp.zeros((_ROWS, 1), jnp.float32)
    for q in range(_QG):
        err_q = jnp.bitwise_and(t[q].astype(jnp.int32), 1).astype(jnp.float32)
        em_q = err_q * maskf[:, 128 * q:128 * (q + 1)]
        s_em = s_em + jnp.sum(em_q)
        row_em = row_em + jnp.sum(em_q, axis=1, keepdims=True)
    fer_cnt = jnp.sum((row_em > 0.0).astype(jnp.float32))

    lane = jax.lax.broadcasted_iota(jnp.int32, (8, 128), 1)
    partial = (jnp.where(lane == 0, s_mask, 0.0)
               + jnp.where(lane == 1, s_em, 0.0)
               + jnp.where(lane == 2, fer_cnt, 0.0))
    out_ref[...] = out_ref[...] + partial


def kernel(info_bits, noise):
    gk = jnp.asarray(_GK_NP, jnp.bfloat16)
    m1 = jnp.asarray(_M1_NP, jnp.float32)
    consts = jnp.asarray(_CONSTS_NP)
    grid = (_BATCH // _ROWS,)
    sums = pl.pallas_call(
        _polar_metrics_kernel,
        grid=grid,
        in_specs=[
            pl.BlockSpec((_ROWS, _K), lambda i: (i, 0)),
            pl.BlockSpec((_ROWS, _N), lambda i: (i, 0)),
            pl.BlockSpec((_K, _N), lambda i: (0, 0)),
            pl.BlockSpec((128, 128), lambda i: (0, 0)),
            pl.BlockSpec((8, _N), lambda i: (0, 0)),
        ],
        out_specs=pl.BlockSpec((8, 128), lambda i: (0, 0)),
        out_shape=jax.ShapeDtypeStruct((8, 128), jnp.float32),
    )(info_bits, noise, gk, m1, consts)
    s_mask = sums[0, 0]
    s_em = sums[0, 1]
    fer_cnt = sums[0, 2]
    b = jnp.float32(_BATCH)
    ber = s_em / jnp.maximum(s_mask, 1.0)
    fer = fer_cnt / b
    rate = s_mask / b
    return (ber, fer, rate)


# speedup vs baseline: 20.9742x; 1.6672x over previous
import numpy as np
import jax
import jax.numpy as jnp
from jax.experimental import pallas as pl

_N = 4096
_K = 2048
_EBNO_DB = 2.0
_THRESH = 0.9
_BATCH = 1024
_ROWS = 128
_QG = _N // 128
_NP = _QG // 2


def _code_construction():
    z = np.array([0.5], dtype=np.float64)
    while z.size < _N:
        z = np.concatenate([2.0 * z - z * z, z * z])
    info_set = np.sort(np.argsort(z)[:_K])
    return info_set, z


_SIGMA = float(np.sqrt(1.0 / (2.0 * (_K / _N) * 10.0 ** (_EBNO_DB / 10.0))))


def _build_constants():
    info_set, z = _code_construction()
    l = np.arange(128)
    g128 = ((l[:, None] & ~l[None, :]) == 0)

    k0 = np.searchsorted(info_set, np.arange(_QG) * 128)
    k1 = np.searchsorted(info_set, (np.arange(_QG) + 1) * 128)
    enc_off = np.zeros((_NP,), np.int64)
    enc_z = np.zeros((_NP, 256, 256), np.float32)
    for p in range(_NP):
        a = min(int(k0[2 * p]), _K - 256)
        enc_off[p] = a
        for q in (2 * p, 2 * p + 1):
            cb = 128 * (q - 2 * p)
            for k in range(int(k0[q]), int(k1[q])):
                r = k - a
                ll = int(info_set[k]) % 128
                enc_z[p, r, cb:cb + 128] = g128[:, ll]

    m1 = g128.T.astype(np.float32)
    m2 = np.zeros((256, 256), np.float32)
    m2[:128, :128] = m1
    m2[128:, 128:] = m1

    r = 1.0 - z
    t = 1.8 - r
    tau = np.full((_N,), np.inf)
    fin = (t > 0.0) & (t < 1.0)
    tau[fin] = (_SIGMA * _SIGMA / 2.0) * np.log(t[fin] / (1.0 - t[fin]))
    tau[t <= 0.0] = -np.inf
    frozen = np.ones((_N,), bool)
    frozen[info_set] = False
    tau[frozen] = np.inf
    consts = np.zeros((8, _N), np.float32)
    consts[0, :] = tau.astype(np.float32)
    return enc_off, enc_z, m2, consts


_ENC_OFF, _ENC_Z_NP, _M2_NP, _CONSTS_NP = _build_constants()


def _polar_metrics_kernel(bits_ref, noise_ref, ez_ref, m2_ref, c_ref, out_ref):
    i = pl.program_id(0)

    @pl.when(i == 0)
    def _init():
        out_ref[...] = jnp.zeros_like(out_ref)

    bits_bf = bits_ref[...].astype(jnp.bfloat16)
    m2 = m2_ref[...]

    w = []
    for p in range(_NP):
        a = int(_ENC_OFF[p])
        wp = jnp.dot(bits_bf[:, a:a + 256], ez_ref[p],
                     preferred_element_type=jnp.float32)
        w.append(wp[:, :128])
        w.append(wp[:, 128:])
    for s in (1, 2, 4, 8, 16):
        for q in range(_QG):
            if q & s == 0:
                w[q] = w[q] + w[q + s]

    e = [None] * _QG
    masks = [None] * _QG
    for q in range(_QG):
        x_int = jnp.bitwise_and(w[q].astype(jnp.int32), 1)
        cmod = (1 - 2 * x_int).astype(jnp.float32)
        y = cmod + _SIGMA * noise_ref[:, 128 * q:128 * (q + 1)]
        tau = c_ref[0:1, 128 * q:128 * (q + 1)]
        masks[q] = (jnp.abs(y) >= tau).astype(jnp.float32)
        xhat = (y < 0).astype(jnp.int32)
        e[q] = jnp.bitwise_xor(xhat, x_int).astype(jnp.bfloat16)

    t = []
    for p in range(_NP):
        ein = jnp.concatenate([e[2 * p], e[2 * p + 1]], axis=1)
        tp = jnp.dot(ein, m2, preferred_element_type=jnp.float32)
        t.append(tp[:, :128])
        t.append(tp[:, 128:])
    for s in (1, 2, 4, 8, 16):
        for q in range(_QG):
            if q & s == 0:
                t[q] = t[q] + t[q + s]

    acc_m = jnp.zeros((_ROWS, 128), jnp.float32)
    acc_em = jnp.zeros((_ROWS, 128), jnp.float32)
    for q in range(_QG):
        err_q = jnp.bitwise_and(t[q].astype(jnp.int32), 1).astype(jnp.float32)
        acc_m = acc_m + masks[q]
        acc_em = acc_em + err_q * masks[q]
    s_mask = jnp.sum(acc_m)
    s_em = jnp.sum(acc_em)
    row_em = jnp.sum(acc_em, axis=1, keepdims=True)
    fer_cnt = jnp.sum((row_em > 0.0).astype(jnp.float32))

    lane = jax.lax.broadcasted_iota(jnp.int32, (8, 128), 1)
    partial = (jnp.where(lane == 0, s_mask, 0.0)
               + jnp.where(lane == 1, s_em, 0.0)
               + jnp.where(lane == 2, fer_cnt, 0.0))
    out_ref[...] = out_ref[...] + partial


def kernel(info_bits, noise):
    ez = jnp.asarray(_ENC_Z_NP, jnp.bfloat16)
    m2 = jnp.asarray(_M2_NP, jnp.bfloat16)
    consts = jnp.asarray(_CONSTS_NP)
    grid = (_BATCH // _ROWS,)
    sums = pl.pallas_call(
        _polar_metrics_kernel,
        grid=grid,
        in_specs=[
            pl.BlockSpec((_ROWS, _K), lambda i: (i, 0)),
            pl.BlockSpec((_ROWS, _N), lambda i: (i, 0)),
            pl.BlockSpec((_NP, 256, 256), lambda i: (0, 0, 0)),
            pl.BlockSpec((256, 256), lambda i: (0, 0)),
            pl.BlockSpec((8, _N), lambda i: (0, 0)),
        ],
        out_specs=pl.BlockSpec((8, 128), lambda i: (0, 0)),
        out_shape=jax.ShapeDtypeStruct((8, 128), jnp.float32),
    )(info_bits, noise, ez, m2, consts)
    s_mask = sums[0, 0]
    s_em = sums[0, 1]
    fer_cnt = sums[0, 2]
    b = jnp.float32(_BATCH)
    ber = s_em / jnp.maximum(s_mask, 1.0)
    fer = fer_cnt / b
    rate = s_mask / b
    return (ber, fer, rate)


# speedup vs baseline: 23.1951x; 1.1059x over previous
import numpy as np
import jax
import jax.numpy as jnp
from jax.experimental import pallas as pl

_N = 4096
_K = 2048
_EBNO_DB = 2.0
_THRESH = 0.9
_BATCH = 1024
_ROWS = 128
_QG = _N // 256


def _code_construction():
    z = np.array([0.5], dtype=np.float64)
    while z.size < _N:
        z = np.concatenate([2.0 * z - z * z, z * z])
    info_set = np.sort(np.argsort(z)[:_K])
    return info_set, z


_SIGMA = float(np.sqrt(1.0 / (2.0 * (_K / _N) * 10.0 ** (_EBNO_DB / 10.0))))


def _build_constants():
    info_set, z = _code_construction()
    l = np.arange(256)
    g256 = ((l[:, None] & ~l[None, :]) == 0)

    k0 = np.searchsorted(info_set, np.arange(_QG) * 256)
    k1 = np.searchsorted(info_set, (np.arange(_QG) + 1) * 256)
    enc_off = np.zeros((_QG,), np.int64)
    enc_z = np.zeros((_QG, 256, 256), np.float32)
    for q in range(_QG):
        a = min(int(k0[q]), _K - 256)
        enc_off[q] = a
        for k in range(int(k0[q]), int(k1[q])):
            ll = int(info_set[k]) % 256
            enc_z[q, k - a, :] = g256[:, ll]

    m256 = g256.T.astype(np.float32)

    r = 1.0 - z
    t = 1.8 - r
    tau = np.full((_N,), np.inf)
    fin = (t > 0.0) & (t < 1.0)
    tau[fin] = (_SIGMA * _SIGMA / 2.0) * np.log(t[fin] / (1.0 - t[fin]))
    tau[t <= 0.0] = -np.inf
    frozen = np.ones((_N,), bool)
    frozen[info_set] = False
    tau[frozen] = np.inf
    consts = np.zeros((8, _N), np.float32)
    consts[0, :] = tau.astype(np.float32)
    return enc_off, enc_z, m256, consts


_ENC_OFF, _ENC_Z_NP, _M256_NP, _CONSTS_NP = _build_constants()


def _polar_metrics_kernel(bits_ref, noise_ref, ez_ref, m_ref, c_ref, out_ref):
    i = pl.program_id(0)

    @pl.when(i == 0)
    def _init():
        out_ref[...] = jnp.zeros_like(out_ref)

    bits_bf = bits_ref[...].astype(jnp.bfloat16)
    m256 = m_ref[...]

    w = []
    for q in range(_QG):
        a = int(_ENC_OFF[q])
        w.append(jnp.dot(bits_bf[:, a:a + 256], ez_ref[q],
                         preferred_element_type=jnp.float32))
    for s in (1, 2, 4, 8):
        for q in range(_QG):
            if q & s == 0:
                w[q] = w[q] + w[q + s]

    e = [None] * _QG
    masks = [None] * _QG
    for q in range(_QG):
        x_int = jnp.bitwise_and(w[q].astype(jnp.int32), 1)
        nbits = jax.lax.bitcast_convert_type(
            noise_ref[:, 256 * q:256 * (q + 1)], jnp.int32)
        sflip = jax.lax.bitcast_convert_type(
            jnp.bitwise_xor(nbits, x_int << 31), jnp.float32)
        v = 1.0 + _SIGMA * sflip
        tau = c_ref[0:1, 256 * q:256 * (q + 1)]
        masks[q] = (jnp.abs(v) >= tau).astype(jnp.float32)
        e[q] = (v < 0.0).astype(jnp.bfloat16)

    t = [jnp.dot(e[q], m256, preferred_element_type=jnp.float32)
         for q in range(_QG)]
    for s in (1, 2, 4, 8):
        for q in range(_QG):
            if q & s == 0:
                t[q] = t[q] + t[q + s]

    acc_m = jnp.zeros((_ROWS, 256), jnp.float32)
    acc_em = jnp.zeros((_ROWS, 256), jnp.float32)
    for q in range(_QG):
        err_q = jnp.bitwise_and(t[q].astype(jnp.int32), 1).astype(jnp.float32)
        acc_m = acc_m + masks[q]
        acc_em = acc_em + err_q * masks[q]
    s_mask = jnp.sum(acc_m)
    s_em = jnp.sum(acc_em)
    row_em = jnp.sum(acc_em, axis=1, keepdims=True)
    fer_cnt = jnp.sum((row_em > 0.0).astype(jnp.float32))

    lane = jax.lax.broadcasted_iota(jnp.int32, (8, 128), 1)
    partial = (jnp.where(lane == 0, s_mask, 0.0)
               + jnp.where(lane == 1, s_em, 0.0)
               + jnp.where(lane == 2, fer_cnt, 0.0))
    out_ref[...] = out_ref[...] + partial


def kernel(info_bits, noise):
    ez = jnp.asarray(_ENC_Z_NP, jnp.bfloat16)
    m256 = jnp.asarray(_M256_NP, jnp.bfloat16)
    consts = jnp.asarray(_CONSTS_NP)
    grid = (_BATCH // _ROWS,)
    sums = pl.pallas_call(
        _polar_metrics_kernel,
        grid=grid,
        in_specs=[
            pl.BlockSpec((_ROWS, _K), lambda i: (i, 0)),
            pl.BlockSpec((_ROWS, _N), lambda i: (i, 0)),
            pl.BlockSpec((_QG, 256, 256), lambda i: (0, 0, 0)),
            pl.BlockSpec((256, 256), lambda i: (0, 0)),
            pl.BlockSpec((8, _N), lambda i: (0, 0)),
        ],
        out_specs=pl.BlockSpec((8, 128), lambda i: (0, 0)),
        out_shape=jax.ShapeDtypeStruct((8, 128), jnp.float32),
    )(info_bits, noise, ez, m256, consts)
    s_mask = sums[0, 0]
    s_em = sums[0, 1]
    fer_cnt = sums[0, 2]
    b = jnp.float32(_BATCH)
    ber = s_em / jnp.maximum(s_mask, 1.0)
    fer = fer_cnt / b
    rate = s_mask / b
    return (ber, fer, rate)


# speedup vs baseline: 23.7498x; 1.0239x over previous
import numpy as np
import jax
import jax.numpy as jnp
from jax.experimental import pallas as pl

_N = 4096
_K = 2048
_EBNO_DB = 2.0
_THRESH = 0.9
_BATCH = 1024
_ROWS = 256
_QG = _N // 256


def _code_construction():
    z = np.array([0.5], dtype=np.float64)
    while z.size < _N:
        z = np.concatenate([2.0 * z - z * z, z * z])
    info_set = np.sort(np.argsort(z)[:_K])
    return info_set, z


_SIGMA = float(np.sqrt(1.0 / (2.0 * (_K / _N) * 10.0 ** (_EBNO_DB / 10.0))))


def _build_constants():
    info_set, z = _code_construction()
    l = np.arange(256)
    g256 = ((l[:, None] & ~l[None, :]) == 0)

    k0 = np.searchsorted(info_set, np.arange(_QG) * 256)
    k1 = np.searchsorted(info_set, (np.arange(_QG) + 1) * 256)
    enc_off = np.zeros((_QG,), np.int64)
    enc_z = np.zeros((_QG, 256, 256), np.float32)
    for q in range(_QG):
        a = min(int(k0[q]), _K - 256)
        enc_off[q] = a
        for k in range(int(k0[q]), int(k1[q])):
            ll = int(info_set[k]) % 256
            enc_z[q, k - a, :] = g256[:, ll]

    m256 = g256.T.astype(np.float32)

    r = 1.0 - z
    t = 1.8 - r
    tau = np.full((_N,), np.inf)
    fin = (t > 0.0) & (t < 1.0)
    tau[fin] = (_SIGMA * _SIGMA / 2.0) * np.log(t[fin] / (1.0 - t[fin]))
    tau[t <= 0.0] = -np.inf
    frozen = np.ones((_N,), bool)
    frozen[info_set] = False
    tau[frozen] = np.inf
    consts = np.zeros((8, _N), np.float32)
    consts[0, :] = tau.astype(np.float32)
    return enc_off, enc_z, m256, consts


_ENC_OFF, _ENC_Z_NP, _M256_NP, _CONSTS_NP = _build_constants()


def _polar_metrics_kernel(bits_ref, noise_ref, ez_ref, m_ref, c_ref, out_ref):
    i = pl.program_id(0)

    @pl.when(i == 0)
    def _init():
        out_ref[...] = jnp.zeros_like(out_ref)

    bits_bf = bits_ref[...].astype(jnp.bfloat16)
    m256 = m_ref[...]

    w = []
    for q in range(_QG):
        a = int(_ENC_OFF[q])
        w.append(jnp.dot(bits_bf[:, a:a + 256], ez_ref[q],
                         preferred_element_type=jnp.float32))
    for s in (1, 2, 4, 8):
        for q in range(_QG):
            if q & s == 0:
                w[q] = w[q] + w[q + s]

    e = [None] * _QG
    masks = [None] * _QG
    for q in range(_QG):
        x_int = jnp.bitwise_and(w[q].astype(jnp.int32), 1)
        nbits = jax.lax.bitcast_convert_type(
            noise_ref[:, 256 * q:256 * (q + 1)], jnp.int32)
        sflip = jax.lax.bitcast_convert_type(
            jnp.bitwise_xor(nbits, x_int << 31), jnp.float32)
        v = 1.0 + _SIGMA * sflip
        tau = c_ref[0:1, 256 * q:256 * (q + 1)]
        masks[q] = (jnp.abs(v) >= tau).astype(jnp.float32)
        e[q] = (v < 0.0).astype(jnp.bfloat16)

    t = [jnp.dot(e[q], m256, preferred_element_type=jnp.float32)
         for q in range(_QG)]
    for s in (1, 2, 4, 8):
        for q in range(_QG):
            if q & s == 0:
                t[q] = t[q] + t[q + s]

    acc_m = jnp.zeros((_ROWS, 256), jnp.float32)
    acc_em = jnp.zeros((_ROWS, 256), jnp.float32)
    for q in range(_QG):
        err_q = jnp.bitwise_and(t[q].astype(jnp.int32), 1).astype(jnp.float32)
        acc_m = acc_m + masks[q]
        acc_em = acc_em + err_q * masks[q]
    s_mask = jnp.sum(acc_m)
    s_em = jnp.sum(acc_em)
    row_em = jnp.sum(acc_em, axis=1, keepdims=True)
    fer_cnt = jnp.sum((row_em > 0.0).astype(jnp.float32))

    lane = jax.lax.broadcasted_iota(jnp.int32, (8, 128), 1)
    partial = (jnp.where(lane == 0, s_mask, 0.0)
               + jnp.where(lane == 1, s_em, 0.0)
               + jnp.where(lane == 2, fer_cnt, 0.0))
    out_ref[...] = out_ref[...] + partial


def kernel(info_bits, noise):
    ez = jnp.asarray(_ENC_Z_NP, jnp.bfloat16)
    m256 = jnp.asarray(_M256_NP, jnp.bfloat16)
    consts = jnp.asarray(_CONSTS_NP)
    grid = (_BATCH // _ROWS,)
    sums = pl.pallas_call(
        _polar_metrics_kernel,
        grid=grid,
        in_specs=[
            pl.BlockSpec((_ROWS, _K), lambda i: (i, 0)),
            pl.BlockSpec((_ROWS, _N), lambda i: (i, 0)),
            pl.BlockSpec((_QG, 256, 256), lambda i: (0, 0, 0)),
            pl.BlockSpec((256, 256), lambda i: (0, 0)),
            pl.BlockSpec((8, _N), lambda i: (0, 0)),
        ],
        out_specs=pl.BlockSpec((8, 128), lambda i: (0, 0)),
        out_shape=jax.ShapeDtypeStruct((8, 128), jnp.float32),
    )(info_bits, noise, ez, m256, consts)
    s_mask = sums[0, 0]
    s_em = sums[0, 1]
    fer_cnt = sums[0, 2]
    b = jnp.float32(_BATCH)
    ber = s_em / jnp.maximum(s_mask, 1.0)
    fer = fer_cnt / b
    rate = s_mask / b
    return (ber, fer, rate)
